# baseline (device time: 13076 ns/iter reference)
import contextlib
import functools
import os

import jax
import jax.numpy as jnp
from jax import lax
from jax.experimental import pallas as pl
from jax.experimental.pallas import tpu as pltpu

M = 1024
D = 512
HALF = M // 2
QUART = M // 4
N_CHUNK = 8
C = QUART // N_CHUNK

_SCOPED = os.environ.get("KERNEL_SCOPES") == "1"


def _scope(name):
    return jax.named_scope(name) if _SCOPED else contextlib.nullcontext()


def kernel(partial, gamma):
    def body(p3_ref, g_ref, out_ref,
             psend, pmine, gv, own_out, peer_out,
             sendy_buf, recvy_buf, sumx_buf, recvx_buf,
             y_send, y_recv, x_send, x_recv, local_sems):
        p_ref = p3_ref.at[0]
        my_x = lax.axis_index("x")
        my_y = lax.axis_index("y")
        ypeer = (my_x, 1 - my_y)
        xpeer = (1 - my_x, my_y)

        with _scope("dma_in_start"):
            send_start = (1 - my_y) * HALF + my_x * QUART
            my_start = my_y * HALF + my_x * QUART
            cp_send = pltpu.make_async_copy(
                p_ref.at[pl.ds(send_start, QUART)], psend, local_sems.at[0])
            cp_mine = pltpu.make_async_copy(
                p_ref.at[pl.ds(my_start, QUART)], pmine, local_sems.at[1])
            cp_g = pltpu.make_async_copy(g_ref, gv, local_sems.at[2])
            cp_send.start()
            cp_mine.start()
            cp_g.start()

        with _scope("barrier"):
            barrier_sem = pltpu.get_barrier_semaphore()
            for nbr in (ypeer, xpeer):
                pl.semaphore_signal(
                    barrier_sem, inc=1,
                    device_id=nbr, device_id_type=pl.DeviceIdType.MESH,
                )
            pl.semaphore_wait(barrier_sem, 2)

        with _scope("cast_and_send_y"):
            cp_send.wait()
            sendy_buf[...] = psend[...].astype(jnp.bfloat16)
            rdma1 = [
                pltpu.make_async_remote_copy(
                    src_ref=sendy_buf.at[pl.ds(k * C, C)],
                    dst_ref=recvy_buf.at[pl.ds(k * C, C)],
                    send_sem=y_send.at[k],
                    recv_sem=y_recv.at[k],
                    device_id=ypeer,
                    device_id_type=pl.DeviceIdType.MESH,
                )
                for k in range(N_CHUNK)
            ]
            for r in rdma1:
                r.start()
            cp_mine.wait()
            cp_g.wait()

        gamma_row = gv[...].astype(jnp.float32).reshape(1, D)
        rdma2 = []
        for k in range(N_CHUNK):
            with _scope(f"hop#k={k}"):
                rdma1[k].wait_recv()
                s_k = (pmine[pl.ds(k * C, C), :]
                       + recvy_buf[pl.ds(k * C, C), :].astype(jnp.float32))
                rms = jnp.sqrt(
                    jnp.mean(s_k * s_k, axis=-1, keepdims=True) + 1e-6)
                n_k = s_k / rms * gamma_row
                own_out[pl.ds(k * C, C), :] = n_k
                sumx_buf[pl.ds(k * C, C), :] = n_k.astype(jnp.bfloat16)
                r2 = pltpu.make_async_remote_copy(
                    src_ref=sumx_buf.at[pl.ds(k * C, C)],
                    dst_ref=recvx_buf.at[pl.ds(k * C, C)],
                    send_sem=x_send.at[k],
                    recv_sem=x_recv.at[k],
                    device_id=xpeer,
                    device_id_type=pl.DeviceIdType.MESH,
                )
                r2.start()
                rdma2.append(r2)

        with _scope("store_own"):
            cp_own = pltpu.make_async_copy(
                own_out, out_ref.at[pl.ds(my_x * QUART, QUART)],
                local_sems.at[0])
            cp_own.start()

        with _scope("wait_x"):
            for r in rdma2:
                r.wait_recv()
        with _scope("store_peer"):
            peer_out[...] = recvx_buf[...].astype(jnp.float32)
            cp_peer = pltpu.make_async_copy(
                peer_out, out_ref.at[pl.ds((1 - my_x) * QUART, QUART)],
                local_sems.at[1])
            cp_peer.start()
            cp_own.wait()
            cp_peer.wait()

        with _scope("drain"):
            for r in rdma1:
                r.wait_send()
            for r in rdma2:
                r.wait_send()

    return pl.pallas_call(
        body,
        out_shape=jax.ShapeDtypeStruct((HALF, D), jnp.float32),
        in_specs=[
            pl.BlockSpec(memory_space=pl.ANY),
            pl.BlockSpec(memory_space=pl.ANY),
        ],
        out_specs=pl.BlockSpec(memory_space=pl.ANY),
        scratch_shapes=[
            pltpu.VMEM((QUART, D), jnp.float32),
            pltpu.VMEM((QUART, D), jnp.float32),
            pltpu.VMEM((D,), jnp.float32),
            pltpu.VMEM((QUART, D), jnp.float32),
            pltpu.VMEM((QUART, D), jnp.float32),
            pltpu.VMEM((QUART, D), jnp.bfloat16),
            pltpu.VMEM((QUART, D), jnp.bfloat16),
            pltpu.VMEM((QUART, D), jnp.bfloat16),
            pltpu.VMEM((QUART, D), jnp.bfloat16),
            pltpu.SemaphoreType.DMA((N_CHUNK,)),
            pltpu.SemaphoreType.DMA((N_CHUNK,)),
            pltpu.SemaphoreType.DMA((N_CHUNK,)),
            pltpu.SemaphoreType.DMA((N_CHUNK,)),
            pltpu.SemaphoreType.DMA((3,)),
        ],
        compiler_params=pltpu.CompilerParams(collective_id=0),
    )(partial, gamma)


# device time: 11551 ns/iter; 1.1320x vs baseline; 1.1320x over previous
import contextlib
import functools
import os

import jax
import jax.numpy as jnp
from jax import lax
from jax.experimental import pallas as pl
from jax.experimental.pallas import tpu as pltpu

M = 1024
D = 512
HALF = M // 2
QUART = M // 4
N_CHUNK = 8
C = QUART // N_CHUNK

_SCOPED = os.environ.get("KERNEL_SCOPES") == "1"


def _scope(name):
    return jax.named_scope(name) if _SCOPED else contextlib.nullcontext()


def kernel(partial, gamma):
    def body(p3_ref, g_ref, out_ref,
             psend, pmine, gv, own_out, peer_out,
             sendy_buf, recvy_buf, sumx_buf, recvx_buf,
             y_send, y_recv, x_send, x_recv, local_sems):
        p_ref = p3_ref.at[0]
        my_x = lax.axis_index("x")
        my_y = lax.axis_index("y")
        ypeer = (my_x, 1 - my_y)
        xpeer = (1 - my_x, my_y)

        with _scope("dma_in_start"):
            send_start = (1 - my_y) * HALF + my_x * QUART
            my_start = my_y * HALF + my_x * QUART
            cp_send = pltpu.make_async_copy(
                p_ref.at[pl.ds(send_start, QUART)], psend, local_sems.at[0])
            cp_mine = pltpu.make_async_copy(
                p_ref.at[pl.ds(my_start, QUART)], pmine, local_sems.at[1])
            cp_g = pltpu.make_async_copy(g_ref, gv, local_sems.at[2])
            cp_send.start()
            cp_mine.start()
            cp_g.start()

        with _scope("barrier"):
            barrier_sem = pltpu.get_barrier_semaphore()
            for nbr in (ypeer, xpeer):
                pl.semaphore_signal(
                    barrier_sem, inc=1,
                    device_id=nbr, device_id_type=pl.DeviceIdType.MESH,
                )
            pl.semaphore_wait(barrier_sem, 2)

        with _scope("cast_and_send_y"):
            cp_send.wait()
            sendy_buf[...] = psend[...].astype(jnp.bfloat16)
            rdma1 = [
                pltpu.make_async_remote_copy(
                    src_ref=sendy_buf.at[pl.ds(k * C, C)],
                    dst_ref=recvy_buf.at[pl.ds(k * C, C)],
                    send_sem=y_send.at[k],
                    recv_sem=y_recv.at[k],
                    device_id=ypeer,
                    device_id_type=pl.DeviceIdType.MESH,
                )
                for k in range(N_CHUNK)
            ]
            for r in rdma1:
                r.start()
            cp_mine.wait()
            cp_g.wait()

        gamma_row = gv[...].astype(jnp.float32).reshape(1, D)
        rdma2 = []
        for k in range(N_CHUNK):
            with _scope(f"hop#k={k}"):
                rdma1[k].wait_recv()
                s_k = (pmine[pl.ds(k * C, C), :]
                       + recvy_buf[pl.ds(k * C, C), :].astype(jnp.float32))
                rms = jnp.sqrt(
                    jnp.mean(s_k * s_k, axis=-1, keepdims=True) + 1e-6)
                n_k = s_k / rms * gamma_row
                own_out[pl.ds(k * C, C), :] = n_k
                sumx_buf[pl.ds(k * C, C), :] = n_k.astype(jnp.bfloat16)
                r2 = pltpu.make_async_remote_copy(
                    src_ref=sumx_buf.at[pl.ds(k * C, C)],
                    dst_ref=recvx_buf.at[pl.ds(k * C, C)],
                    send_sem=x_send.at[k],
                    recv_sem=x_recv.at[k],
                    device_id=xpeer,
                    device_id_type=pl.DeviceIdType.MESH,
                )
                r2.start()
                rdma2.append(r2)

        with _scope("store_own"):
            cp_own = pltpu.make_async_copy(
                own_out, out_ref.at[pl.ds(my_x * QUART, QUART)],
                local_sems.at[0])
            cp_own.start()

        with _scope("wait_x"):
            for r in rdma2:
                r.wait_recv()
        with _scope("store_peer"):
            peer_out[...] = recvx_buf[...].astype(jnp.float32)
            cp_peer = pltpu.make_async_copy(
                peer_out, out_ref.at[pl.ds((1 - my_x) * QUART, QUART)],
                local_sems.at[1])
            cp_peer.start()
            cp_own.wait()
            cp_peer.wait()

        with _scope("drain"):
            for r in rdma1:
                r.wait_send()
            for r in rdma2:
                r.wait_send()

    return pl.pallas_call(
        body,
        out_shape=jax.ShapeDtypeStruct((HALF, D), jnp.float32),
        in_specs=[
            pl.BlockSpec(memory_space=pl.ANY),
            pl.BlockSpec(memory_space=pl.ANY),
        ],
        out_specs=pl.BlockSpec(memory_space=pl.ANY),
        scratch_shapes=[
            pltpu.VMEM((QUART, D), jnp.float32),
            pltpu.VMEM((QUART, D), jnp.float32),
            pltpu.VMEM((D,), jnp.float32),
            pltpu.VMEM((QUART, D), jnp.float32),
            pltpu.VMEM((QUART, D), jnp.float32),
            pltpu.VMEM((QUART, D), jnp.bfloat16),
            pltpu.VMEM((QUART, D), jnp.bfloat16),
            pltpu.VMEM((QUART, D), jnp.bfloat16),
            pltpu.VMEM((QUART, D), jnp.bfloat16),
            pltpu.SemaphoreType.DMA((N_CHUNK,)),
            pltpu.SemaphoreType.DMA((N_CHUNK,)),
            pltpu.SemaphoreType.DMA((N_CHUNK,)),
            pltpu.SemaphoreType.DMA((N_CHUNK,)),
            pltpu.SemaphoreType.DMA((3,)),
        ],
        compiler_params=pltpu.CompilerParams(collective_id=0),
    )(
        pltpu.with_memory_space_constraint(partial, pltpu.MemorySpace.HBM),
        pltpu.with_memory_space_constraint(gamma, pltpu.MemorySpace.HBM),
    )


# device time: 11544 ns/iter; 1.1327x vs baseline; 1.0006x over previous
import contextlib
import functools
import os

import jax
import jax.numpy as jnp
from jax import lax
from jax.experimental import pallas as pl
from jax.experimental.pallas import tpu as pltpu

M = 1024
D = 512
HALF = M // 2
QUART = M // 4
N_CHUNK = 8
C = QUART // N_CHUNK

_SCOPED = os.environ.get("KERNEL_SCOPES") == "1"


def _scope(name):
    return jax.named_scope(name) if _SCOPED else contextlib.nullcontext()


def kernel(partial, gamma):
    def body(p3_ref, g_ref, out_ref,
             psend, pmine, gv, own_out, peer_out,
             sendy_buf, recvy_buf, sumx_buf, recvx_buf,
             y_send, y_recv, x_send, x_recv, local_sems):
        p_ref = p3_ref.at[0]
        my_x = lax.axis_index("x")
        my_y = lax.axis_index("y")
        ypeer = (my_x, 1 - my_y)
        xpeer = (1 - my_x, my_y)

        with _scope("dma_in_start"):
            send_start = (1 - my_y) * HALF + my_x * QUART
            my_start = my_y * HALF + my_x * QUART
            cp_send = pltpu.make_async_copy(
                p_ref.at[pl.ds(send_start, QUART)], psend, local_sems.at[0])
            cp_mine = pltpu.make_async_copy(
                p_ref.at[pl.ds(my_start, QUART)], pmine, local_sems.at[1])
            cp_g = pltpu.make_async_copy(g_ref, gv, local_sems.at[2])
            cp_send.start()
            cp_mine.start()
            cp_g.start()

        with _scope("barrier"):
            barrier_sem = pltpu.get_barrier_semaphore()
            for nbr in (ypeer, xpeer):
                pl.semaphore_signal(
                    barrier_sem, inc=1,
                    device_id=nbr, device_id_type=pl.DeviceIdType.MESH,
                )
            pl.semaphore_wait(barrier_sem, 2)

        with _scope("cast_and_send_y"):
            cp_send.wait()
            sendy_buf[...] = psend[...].astype(jnp.bfloat16)
            rdma1 = [
                pltpu.make_async_remote_copy(
                    src_ref=sendy_buf.at[pl.ds(k * C, C)],
                    dst_ref=recvy_buf.at[pl.ds(k * C, C)],
                    send_sem=y_send.at[k],
                    recv_sem=y_recv.at[k],
                    device_id=ypeer,
                    device_id_type=pl.DeviceIdType.MESH,
                )
                for k in range(N_CHUNK)
            ]
            for r in rdma1:
                r.start()
            cp_mine.wait()
            cp_g.wait()

        gamma_row = gv[...].astype(jnp.float32).reshape(1, D)
        rdma2 = []
        for k in range(N_CHUNK):
            with _scope(f"hop#k={k}"):
                rdma1[k].wait_recv()
                s_k = (pmine[pl.ds(k * C, C), :]
                       + recvy_buf[pl.ds(k * C, C), :].astype(jnp.float32))
                rms = jnp.sqrt(
                    jnp.mean(s_k * s_k, axis=-1, keepdims=True) + 1e-6)
                n_k = s_k / rms * gamma_row
                own_out[pl.ds(k * C, C), :] = n_k
                sumx_buf[pl.ds(k * C, C), :] = n_k.astype(jnp.bfloat16)
                r2 = pltpu.make_async_remote_copy(
                    src_ref=sumx_buf.at[pl.ds(k * C, C)],
                    dst_ref=recvx_buf.at[pl.ds(k * C, C)],
                    send_sem=x_send.at[k],
                    recv_sem=x_recv.at[k],
                    device_id=xpeer,
                    device_id_type=pl.DeviceIdType.MESH,
                )
                r2.start()
                rdma2.append(r2)

        with _scope("store_own"):
            cp_own = pltpu.make_async_copy(
                own_out, out_ref.at[pl.ds(my_x * QUART, QUART)],
                local_sems.at[0])
            cp_own.start()

        with _scope("wait_x"):
            for r in rdma2:
                r.wait_recv()
        with _scope("store_peer"):
            peer_out[...] = recvx_buf[...].astype(jnp.float32)
            cp_peer = pltpu.make_async_copy(
                peer_out, out_ref.at[pl.ds((1 - my_x) * QUART, QUART)],
                local_sems.at[1])
            cp_peer.start()
            cp_own.wait()
            cp_peer.wait()

        with _scope("drain"):
            for r in rdma1:
                r.wait_send()
            for r in rdma2:
                r.wait_send()

    return pl.pallas_call(
        body,
        out_shape=jax.ShapeDtypeStruct((HALF, D), jnp.float32),
        in_specs=[
            pl.BlockSpec(memory_space=pl.ANY),
            pl.BlockSpec(memory_space=pl.ANY),
        ],
        out_specs=pl.BlockSpec(memory_space=pltpu.MemorySpace.HBM),
        scratch_shapes=[
            pltpu.VMEM((QUART, D), jnp.float32),
            pltpu.VMEM((QUART, D), jnp.float32),
            pltpu.VMEM((D,), jnp.float32),
            pltpu.VMEM((QUART, D), jnp.float32),
            pltpu.VMEM((QUART, D), jnp.float32),
            pltpu.VMEM((QUART, D), jnp.bfloat16),
            pltpu.VMEM((QUART, D), jnp.bfloat16),
            pltpu.VMEM((QUART, D), jnp.bfloat16),
            pltpu.VMEM((QUART, D), jnp.bfloat16),
            pltpu.SemaphoreType.DMA((N_CHUNK,)),
            pltpu.SemaphoreType.DMA((N_CHUNK,)),
            pltpu.SemaphoreType.DMA((N_CHUNK,)),
            pltpu.SemaphoreType.DMA((N_CHUNK,)),
            pltpu.SemaphoreType.DMA((3,)),
        ],
        compiler_params=pltpu.CompilerParams(collective_id=0),
    )(
        pltpu.with_memory_space_constraint(partial, pltpu.MemorySpace.HBM),
        pltpu.with_memory_space_constraint(gamma, pltpu.MemorySpace.HBM),
    )
